# baseline (speedup 1.0000x reference)
import jax
import jax.numpy as jnp
from jax.experimental import pallas as pl
from jax.experimental.pallas import tpu as pltpu

_P = 68
_NROW = _P * _P
_PAD = 72
_NPADROW = _NROW + 2 * _PAD


def _fused_kernel(o0_ref, o1_ref, o2_ref, o3_ref, o4_ref,
                  w1_ref, w2fc_ref, fcb_ref, out_ref, y1_ref, a_ref):
    b = pl.program_id(0)
    d = pl.program_id(1)
    o_refs = (o0_ref, o1_ref, o2_ref, o3_ref, o4_ref)

    @pl.when(jnp.logical_and(b == 0, d == 0))
    def _init():
        y1_ref[...] = jnp.zeros_like(y1_ref)

    for dzi, dz in enumerate((-1, 0, 1)):
        for bb in (-1, 0, 1):
            for cc in (-1, 0, 1):
                s = _P * bb + cc
                term = None
                for a in (-1, 0, 1):
                    tap = ((a + 1) * 3 + (bb + 1)) * 3 + (cc + 1)
                    osl = o_refs[dz + a + 2][0, 0, pl.ds(_PAD + s, _NROW), :]
                    t = osl.astype(jnp.float32) * w1_ref[tap, :][None, :]
                    term = t if term is None else term + t
                if bb == -1 and cc == -1:
                    y1_ref[dzi, _PAD:_PAD + _NROW, :] = term
                else:
                    y1_ref[dzi, _PAD:_PAD + _NROW, :] += term
        y1_ref[dzi, _PAD:_PAD + _NROW, :] *= (
            o_refs[dz + 2][0, 0, _PAD:_PAD + _NROW, :].astype(jnp.float32))

    for dzi in range(3):
        for bb in (-1, 0, 1):
            for cc in (-1, 0, 1):
                g = (dzi * 3 + (bb + 1)) * 3 + (cc + 1)
                s = _P * bb + cc
                a_ref[:, 16 * g:16 * g + 16] = y1_ref[
                    dzi, pl.ds(_PAD + s, _NROW), :].astype(jnp.bfloat16)

    u = jnp.dot(a_ref[:, :], w2fc_ref[:, :],
                preferred_element_type=jnp.float32)
    mask10 = o2_ref[0, 0, _PAD:_PAD + _NROW, 0:10].astype(jnp.float32)
    out_ref[0, 0, :, :] = (u + fcb_ref[0, :][None, :]) * mask10


def kernel(occ, w1, w2, fc_w, fc_b):
    B, D, H, W = occ.shape
    o = occ.astype(jnp.bfloat16)
    o_pad = jnp.pad(o, ((0, 0), (2, 2), (2, 2), (2, 2)))
    o_flat = o_pad.reshape(B, D + 4, _NROW, 1)
    o_flat = jnp.pad(o_flat, ((0, 0), (0, 0), (_PAD, _PAD), (0, 0)))
    o_rep = jnp.broadcast_to(o_flat, (B, D + 4, _NPADROW, 16))
    w1r = w1.reshape(27, 16)
    w2fc = (w2.reshape(432, 32) @ fc_w.T).astype(jnp.bfloat16)
    fcb = fc_b.reshape(1, 10)

    oblk = (1, 1, _NPADROW, 16)
    out = pl.pallas_call(
        _fused_kernel,
        grid=(B, D),
        in_specs=[
            pl.BlockSpec(oblk, lambda b, d: (b, d, 0, 0)),
            pl.BlockSpec(oblk, lambda b, d: (b, d + 1, 0, 0)),
            pl.BlockSpec(oblk, lambda b, d: (b, d + 2, 0, 0)),
            pl.BlockSpec(oblk, lambda b, d: (b, d + 3, 0, 0)),
            pl.BlockSpec(oblk, lambda b, d: (b, d + 4, 0, 0)),
            pl.BlockSpec(w1r.shape, lambda b, d: (0, 0)),
            pl.BlockSpec(w2fc.shape, lambda b, d: (0, 0)),
            pl.BlockSpec(fcb.shape, lambda b, d: (0, 0)),
        ],
        out_specs=pl.BlockSpec((1, 1, _NROW, 10), lambda b, d: (b, d, 0, 0)),
        out_shape=jax.ShapeDtypeStruct((B, D, _NROW, 10), jnp.float32),
        scratch_shapes=[
            pltpu.VMEM((3, _NPADROW, 16), jnp.float32),
            pltpu.VMEM((_NROW, 432), jnp.bfloat16),
        ],
    )(o_rep, o_rep, o_rep, o_rep, o_rep, w1r, w2fc, fcb)
    out = out.reshape(B, D, _P, _P, 10)[:, :, 2:2 + H, 2:2 + W, :]
    return out.reshape(B * D * H * W, 10)
